# Initial kernel scaffold; baseline (speedup 1.0000x reference)
import jax
import jax.numpy as jnp
from jax.experimental import pallas as pl


def kernel(pos, batch, params):
    raise NotImplementedError("write your pallas kernel here")


# speedup vs baseline: 10.9044x; 10.9044x over previous
import functools

import jax
import jax.numpy as jnp
from jax import lax
from jax.experimental import pallas as pl
from jax.experimental.pallas import tpu as pltpu
from jax.experimental.pallas import tpu_sc as plsc

B = 8
P = 2048
K1 = 1024
K2 = 256
NB = 64
R1SQ = 0.2 * 0.2
R2SQ = 0.4 * 0.4

_NC = 2
_NS = 16
_NW = _NC * _NS

_N1 = B * K1 * NB
_N2 = B * K2 * NB


def _fps_phase(px, py, pz, k, out_ref):
    n = px.shape[1]
    iota = lax.broadcasted_iota(jnp.int32, (B, n), 1)
    kiota = lax.broadcasted_iota(jnp.int32, (B, k), 1)
    cx = px[:, 0:1]
    cy = py[:, 0:1]
    cz = pz[:, 0:1]
    zk = jnp.zeros((B, k), jnp.float32)
    sel0 = kiota == 0
    pdx = jnp.where(sel0, cx, zk)
    pdy = jnp.where(sel0, cy, zk)
    pdz = jnp.where(sel0, cz, zk)

    def body(i, st):
        d, cx, cy, cz, pdx, pdy, pdz = st
        dn = (px - cx) ** 2 + (py - cy) ** 2 + (pz - cz) ** 2
        d = jnp.minimum(d, dn)
        mx = jnp.max(d, axis=1, keepdims=True)
        win = d == mx
        idx = jnp.min(jnp.where(win, iota, n), axis=1, keepdims=True)
        sel = iota == idx
        cx = jnp.max(jnp.where(sel, px, -jnp.inf), axis=1, keepdims=True)
        cy = jnp.max(jnp.where(sel, py, -jnp.inf), axis=1, keepdims=True)
        cz = jnp.max(jnp.where(sel, pz, -jnp.inf), axis=1, keepdims=True)
        selk = kiota == i
        pdx = jnp.where(selk, cx, pdx)
        pdy = jnp.where(selk, cy, pdy)
        pdz = jnp.where(selk, cz, pdz)
        return d, cx, cy, cz, pdx, pdy, pdz

    d0 = jnp.full((B, n), jnp.inf, jnp.float32)
    st = lax.fori_loop(1, k, body, (d0, cx, cy, cz, pdx, pdy, pdz))
    out_ref[0] = st[4]
    out_ref[1] = st[5]
    out_ref[2] = st[6]


def _fps_body(posT_ref, pd1_ref, pd2_ref):
    _fps_phase(posT_ref[0], posT_ref[1], posT_ref[2], K1, pd1_ref)
    _fps_phase(pd1_ref[0], pd1_ref[1], pd1_ref[2], K2, pd2_ref)


def _bcast(vec, t):
    idx = jnp.full((16, 1), t, jnp.int32)
    dnums = lax.GatherDimensionNumbers(
        offset_dims=(), collapsed_slice_dims=(0,), start_index_map=(0,))
    return lax.gather(vec, idx, dnums, slice_sizes=(1,),
                      mode=lax.GatherScatterMode.PROMISE_IN_BOUNDS)


def _ball_query(px_v, py_v, pz_v, dx_v, dy_v, dz_v, cols_v, relbuf_v,
                n_src, n_groups, n_chunks, r2):
    iota16 = lax.iota(jnp.int32, 16)
    r2f = jnp.float32(r2)

    def group_body(g, _):
        gdx = dx_v[pl.ds(g * 16, 16)]
        gdy = dy_v[pl.ds(g * 16, 16)]
        gdz = dz_v[pl.ds(g * 16, 16)]
        slot0 = (g * 16 + iota16) * NB

        def chunk_body(c, cnt):
            sx = px_v[pl.ds(c * 16, 16)]
            sy = py_v[pl.ds(c * 16, 16)]
            sz = pz_v[pl.ds(c * 16, 16)]
            for t in range(16):
                bx = _bcast(sx, t)
                by = _bcast(sy, t)
                bz = _bcast(sz, t)
                ddx = gdx - bx
                ddy = gdy - by
                ddz = gdz - bz
                d2 = ddx * ddx + ddy * ddy + ddz * ddz
                m = (d2 <= r2f) & (cnt < NB)
                jval = c * 16 + t
                plsc.store_scatter(cols_v, [slot0 + cnt],
                                   jnp.full((16,), jval, jnp.int32), mask=m)
                cnt = cnt + m.astype(jnp.int32)
            return cnt

        cnt = lax.fori_loop(0, n_chunks, chunk_body,
                            jnp.zeros((16,), jnp.int32))

        def emit_body(mm, _):
            mvec = mm * 16 + iota16
            for t in range(16):
                cvec = cols_v[pl.ds((g * 16 + t) * NB + mm * 16, 16)]
                gx = plsc.load_gather(px_v, [cvec])
                gy = plsc.load_gather(py_v, [cvec])
                gz = plsc.load_gather(pz_v, [cvec])
                bx = _bcast(gdx, t)
                by = _bcast(gdy, t)
                bz = _bcast(gdz, t)
                cntt = _bcast(cnt, t)
                val = (mvec < cntt).astype(jnp.float32)
                e4 = (((g * 16 + t) * NB) + mvec) * 4
                plsc.store_scatter(relbuf_v, [e4], gx - bx)
                plsc.store_scatter(relbuf_v, [e4 + 1], gy - by)
                plsc.store_scatter(relbuf_v, [e4 + 2], gz - bz)
                plsc.store_scatter(relbuf_v, [e4 + 3], val)
            return 0

        lax.fori_loop(0, NB // 16, emit_body, 0)
        return 0

    lax.fori_loop(0, n_groups, group_body, 0)


def _zero_i32(ref, n):
    z = jnp.zeros((16,), jnp.int32)

    def zi(i, _):
        ref[pl.ds(i * 16, 16)] = z
        return 0

    lax.fori_loop(0, n // 16, zi, 0)


def _nbr1_body(posT_hbm, pd1_hbm, rel_hbm,
               px_v, py_v, pz_v, dx_v, dy_v, dz_v, cols_v, relbuf_v):
    wid = lax.axis_index("c") * _NS + lax.axis_index("s")
    cloud = wid // 4
    part = wid % 4
    pltpu.sync_copy(posT_hbm.at[0, cloud], px_v)
    pltpu.sync_copy(posT_hbm.at[1, cloud], py_v)
    pltpu.sync_copy(posT_hbm.at[2, cloud], pz_v)
    pltpu.sync_copy(pd1_hbm.at[0, cloud, pl.ds(part * 256, 256)], dx_v)
    pltpu.sync_copy(pd1_hbm.at[1, cloud, pl.ds(part * 256, 256)], dy_v)
    pltpu.sync_copy(pd1_hbm.at[2, cloud, pl.ds(part * 256, 256)], dz_v)
    _zero_i32(cols_v, 256 * NB)
    _ball_query(px_v, py_v, pz_v, dx_v, dy_v, dz_v, cols_v, relbuf_v,
                P, 16, P // 16, R1SQ)
    pltpu.sync_copy(relbuf_v, rel_hbm.at[pl.ds(wid * (256 * NB * 4),
                                               256 * NB * 4)])


def _nbr2_body(pd1_hbm, pd2_hbm, x1_hbm, rel_hbm, xj_hbm,
               px_v, py_v, pz_v, dx_v, dy_v, dz_v, cols_v, relbuf_v,
               gidx_v, rows_v, sem):
    wid = lax.axis_index("c") * _NS + lax.axis_index("s")
    cloud = wid // 4
    part = wid % 4
    pltpu.sync_copy(pd1_hbm.at[0, cloud], px_v)
    pltpu.sync_copy(pd1_hbm.at[1, cloud], py_v)
    pltpu.sync_copy(pd1_hbm.at[2, cloud], pz_v)
    pltpu.sync_copy(pd2_hbm.at[0, cloud, pl.ds(part * 64, 64)], dx_v)
    pltpu.sync_copy(pd2_hbm.at[1, cloud, pl.ds(part * 64, 64)], dy_v)
    pltpu.sync_copy(pd2_hbm.at[2, cloud, pl.ds(part * 64, 64)], dz_v)
    _zero_i32(cols_v, 64 * NB)
    _ball_query(px_v, py_v, pz_v, dx_v, dy_v, dz_v, cols_v, relbuf_v,
                K1, 4, K1 // 16, R2SQ)
    pltpu.sync_copy(relbuf_v, rel_hbm.at[pl.ds(wid * (64 * NB * 4),
                                               64 * NB * 4)])
    cbase = cloud * K1

    def gath_body(ch, _):
        for i in range(8):
            cv = cols_v[pl.ds(ch * 128 + i * 16, 16)]
            gidx_v[pl.ds(i * 16, 16)] = cv + cbase
        pltpu.async_copy(x1_hbm.at[gidx_v], rows_v, sem).wait()
        pltpu.sync_copy(rows_v, xj_hbm.at[pl.ds(wid * 4096 + ch * 128, 128)])
        return 0

    lax.fori_loop(0, 32, gath_body, 0)


def _conv1_body(rel_ref, w1_ref, b1_ref, w2_ref, b2_ref, w3_ref, b3_ref,
                out_ref):
    relb = rel_ref[...]
    h = jnp.dot(relb, w1_ref[...], preferred_element_type=jnp.float32)
    h = jnp.maximum(h + b1_ref[...], 0.0)
    h = jnp.dot(h, w2_ref[...], preferred_element_type=jnp.float32)
    h = jnp.maximum(h + b2_ref[...], 0.0)
    h = jnp.dot(h, w3_ref[...], preferred_element_type=jnp.float32)
    h = h + b3_ref[...]
    h = jnp.where(relb[:, 3:4] > 0.5, h, -jnp.inf)
    out_ref[...] = jnp.max(h.reshape(16, NB, 128), axis=1)


def _conv2_body(xj_ref, rel_ref, wx_ref, wr_ref, b1_ref, w2_ref, b2_ref,
                w3_ref, b3_ref, out_ref):
    relb = rel_ref[...]
    h = (jnp.dot(xj_ref[...], wx_ref[...], preferred_element_type=jnp.float32)
         + jnp.dot(relb, wr_ref[...], preferred_element_type=jnp.float32))
    h = jnp.maximum(h + b1_ref[...], 0.0)
    h = jnp.dot(h, w2_ref[...], preferred_element_type=jnp.float32)
    h = jnp.maximum(h + b2_ref[...], 0.0)
    h = jnp.dot(h, w3_ref[...], preferred_element_type=jnp.float32)
    h = h + b3_ref[...]
    h = jnp.where(relb[:, 3:4] > 0.5, h, -jnp.inf)
    out_ref[...] = jnp.max(h.reshape(8, NB, 256), axis=1)


def _tail_body(x2_ref, p2_ref, wx_ref, wr_ref, b1_ref, w2_ref, b2_ref,
               w3_ref, b3_ref, hw1_ref, hb1_ref, hw2_ref, hb2_ref,
               hw3_ref, hb3_ref, out_ref):
    h = (jnp.dot(x2_ref[...], wx_ref[...], preferred_element_type=jnp.float32)
         + jnp.dot(p2_ref[...], wr_ref[...], preferred_element_type=jnp.float32))
    h = jnp.maximum(h + b1_ref[...], 0.0)
    h = jnp.dot(h, w2_ref[...], preferred_element_type=jnp.float32)
    h = jnp.maximum(h + b2_ref[...], 0.0)
    h = jnp.dot(h, w3_ref[...], preferred_element_type=jnp.float32)
    h = h + b3_ref[...]
    g = jnp.max(h, axis=0, keepdims=True)
    g = jnp.maximum(jnp.dot(g, hw1_ref[...],
                            preferred_element_type=jnp.float32) + hb1_ref[...],
                    0.0)
    g = jnp.maximum(jnp.dot(g, hw2_ref[...],
                            preferred_element_type=jnp.float32) + hb2_ref[...],
                    0.0)
    row = jnp.dot(g, hw3_ref[...],
                  preferred_element_type=jnp.float32) + hb3_ref[...]
    out_ref[pl.ds(pl.program_id(0), 1), :] = row


def _fold_bn(p):
    ws, bs = [], []
    n = len(p["W"])
    s_denom = jnp.sqrt(jnp.float32(1.0 + 1e-5))
    for i in range(n):
        w, b = p["W"][i], p["b"][i]
        if i < n - 1:
            s = p["g"][i] / s_denom
            ws.append(w * s[None, :])
            bs.append((b * s + p["beta"][i])[None, :])
        else:
            ws.append(w)
            bs.append(b[None, :])
    return ws, bs


def _pad_rows(w, rows):
    return jnp.concatenate(
        [w, jnp.zeros((rows - w.shape[0], w.shape[1]), w.dtype)], axis=0)


_fps_call = pl.pallas_call(
    _fps_body,
    out_shape=[jax.ShapeDtypeStruct((3, B, K1), jnp.float32),
               jax.ShapeDtypeStruct((3, B, K2), jnp.float32)],
)

@functools.lru_cache(maxsize=None)
def _sc_calls():
    mesh = plsc.VectorSubcoreMesh(core_axis_name="c", subcore_axis_name="s",
                                  num_cores=_NC, num_subcores=_NS)
    nbr1 = pl.kernel(
        _nbr1_body,
        out_type=jax.ShapeDtypeStruct((_N1 * 4,), jnp.float32),
        mesh=mesh,
        compiler_params=pltpu.CompilerParams(needs_layout_passes=False),
        scratch_types=[
            pltpu.VMEM((P,), jnp.float32),
            pltpu.VMEM((P,), jnp.float32),
            pltpu.VMEM((P,), jnp.float32),
            pltpu.VMEM((256,), jnp.float32),
            pltpu.VMEM((256,), jnp.float32),
            pltpu.VMEM((256,), jnp.float32),
            pltpu.VMEM((256 * NB,), jnp.int32),
            pltpu.VMEM((256 * NB * 4,), jnp.float32),
        ],
    )
    nbr2 = pl.kernel(
        _nbr2_body,
        out_type=[jax.ShapeDtypeStruct((_N2 * 4,), jnp.float32),
                  jax.ShapeDtypeStruct((_N2, 128), jnp.float32)],
        mesh=mesh,
        compiler_params=pltpu.CompilerParams(needs_layout_passes=False),
        scratch_types=[
            pltpu.VMEM((K1,), jnp.float32),
            pltpu.VMEM((K1,), jnp.float32),
            pltpu.VMEM((K1,), jnp.float32),
            pltpu.VMEM((64,), jnp.float32),
            pltpu.VMEM((64,), jnp.float32),
            pltpu.VMEM((64,), jnp.float32),
            pltpu.VMEM((64 * NB,), jnp.int32),
            pltpu.VMEM((64 * NB * 4,), jnp.float32),
            pltpu.VMEM((128,), jnp.int32),
            pltpu.VMEM((128, 128), jnp.float32),
            pltpu.SemaphoreType.DMA,
        ],
    )
    return nbr1, nbr2

_conv1_call = pl.pallas_call(
    _conv1_body,
    grid=(_N1 // 1024,),
    in_specs=[
        pl.BlockSpec((1024, 4), lambda i: (i, 0)),
        pl.BlockSpec((4, 64), lambda i: (0, 0)),
        pl.BlockSpec((1, 64), lambda i: (0, 0)),
        pl.BlockSpec((64, 64), lambda i: (0, 0)),
        pl.BlockSpec((1, 64), lambda i: (0, 0)),
        pl.BlockSpec((64, 128), lambda i: (0, 0)),
        pl.BlockSpec((1, 128), lambda i: (0, 0)),
    ],
    out_specs=pl.BlockSpec((16, 128), lambda i: (i, 0)),
    out_shape=jax.ShapeDtypeStruct((B * K1, 128), jnp.float32),
)

_conv2_call = pl.pallas_call(
    _conv2_body,
    grid=(_N2 // 512,),
    in_specs=[
        pl.BlockSpec((512, 128), lambda i: (i, 0)),
        pl.BlockSpec((512, 4), lambda i: (i, 0)),
        pl.BlockSpec((128, 128), lambda i: (0, 0)),
        pl.BlockSpec((4, 128), lambda i: (0, 0)),
        pl.BlockSpec((1, 128), lambda i: (0, 0)),
        pl.BlockSpec((128, 128), lambda i: (0, 0)),
        pl.BlockSpec((1, 128), lambda i: (0, 0)),
        pl.BlockSpec((128, 256), lambda i: (0, 0)),
        pl.BlockSpec((1, 256), lambda i: (0, 0)),
    ],
    out_specs=pl.BlockSpec((8, 256), lambda i: (i, 0)),
    out_shape=jax.ShapeDtypeStruct((B * K2, 256), jnp.float32),
)

_tail_call = pl.pallas_call(
    _tail_body,
    grid=(B,),
    in_specs=[
        pl.BlockSpec((K2, 256), lambda i: (i, 0)),
        pl.BlockSpec((K2, 4), lambda i: (i, 0)),
        pl.BlockSpec((256, 256), lambda i: (0, 0)),
        pl.BlockSpec((4, 256), lambda i: (0, 0)),
        pl.BlockSpec((1, 256), lambda i: (0, 0)),
        pl.BlockSpec((256, 512), lambda i: (0, 0)),
        pl.BlockSpec((1, 512), lambda i: (0, 0)),
        pl.BlockSpec((512, 1024), lambda i: (0, 0)),
        pl.BlockSpec((1, 1024), lambda i: (0, 0)),
        pl.BlockSpec((1024, 512), lambda i: (0, 0)),
        pl.BlockSpec((1, 512), lambda i: (0, 0)),
        pl.BlockSpec((512, 256), lambda i: (0, 0)),
        pl.BlockSpec((1, 256), lambda i: (0, 0)),
        pl.BlockSpec((256, 64), lambda i: (0, 0)),
        pl.BlockSpec((1, 64), lambda i: (0, 0)),
    ],
    out_specs=pl.BlockSpec((B, 64), lambda i: (0, 0)),
    out_shape=jax.ShapeDtypeStruct((B, 64), jnp.float32),
)


def kernel(pos, batch, params):
    posT = jnp.transpose(pos.reshape(B, P, 3), (2, 0, 1))
    pd1, pd2 = _fps_call(posT)

    _nbr1_call, _nbr2_call = _sc_calls()
    rel1 = _nbr1_call(posT, pd1).reshape(_N1, 4)

    w1, b1 = _fold_bn(params["sa1"])
    x1 = _conv1_call(rel1, _pad_rows(w1[0], 4), b1[0],
                     w1[1], b1[1], w1[2], b1[2])

    rel2, xj2 = _nbr2_call(pd1, pd2, x1)
    rel2 = rel2.reshape(_N2, 4)

    w2, b2 = _fold_bn(params["sa2"])
    x2 = _conv2_call(xj2, rel2, w2[0][:128], _pad_rows(w2[0][128:], 4),
                     b2[0], w2[1], b2[1], w2[2], b2[2])

    w3, b3 = _fold_bn(params["sa3"])
    wh = params["head"]["W"]
    bh = [b[None, :] for b in params["head"]["b"]]
    p2 = jnp.concatenate(
        [jnp.transpose(pd2, (1, 2, 0)).reshape(B * K2, 3),
         jnp.zeros((B * K2, 1), jnp.float32)], axis=1)
    out = _tail_call(x2, p2, w3[0][:256], _pad_rows(w3[0][256:], 4),
                     b3[0], w3[1], b3[1], w3[2], b3[2],
                     wh[0], bh[0], wh[1], bh[1], wh[2], bh[2])
    return out
